# Initial kernel scaffold; baseline (speedup 1.0000x reference)
import jax
import jax.numpy as jnp
from jax.experimental import pallas as pl


def kernel(x, edge_index, batch, W1, b1, W2, b2):
    raise NotImplementedError("write your pallas kernel here")


# speedup vs baseline: 22.3454x; 22.3454x over previous
import functools

import jax
import jax.numpy as jnp
from jax import lax
from jax.experimental import pallas as pl
from jax.experimental.pallas import tpu as pltpu
from jax.experimental.pallas import tpu_sc as plsc

N = 10000
NPAD = 10240
D = 128
E = 320000
G = 64
NC, NS = 2, 16
NW = NC * NS
EPW = E // NW
K = 125
NCH = EPW // K
RPT = NPAD // NS
WPT = NPAD // NS
ZR = 160
BLK = 1000
GRID = N // BLK

@functools.cache
def _mesh():
    return plsc.VectorSubcoreMesh(core_axis_name="c", subcore_axis_name="s",
                                  num_cores=NC, num_subcores=NS)


def _deg_body(col3, out, col_v, ones_v, zrow_v, acc_sh):
    c = lax.axis_index("c")
    s = lax.axis_index("s")
    wid = c * NS + s

    for i in range(8):
        ones_v[pl.ds(16 * i, 16)] = jnp.ones((16,), jnp.float32)
    for i in range(WPT // 16):
        zrow_v[pl.ds(16 * i, 16)] = jnp.zeros((16,), jnp.float32)
    pltpu.sync_copy(zrow_v, acc_sh.at[pl.ds(s * WPT, WPT)])
    plsc.subcore_barrier()

    pltpu.sync_copy(col3.at[wid], col_v)

    def chunk(j, carry):
        pltpu.sync_copy(ones_v.at[pl.ds(0, K)], acc_sh.at[col_v.at[j]],
                        add=True)
        return carry

    lax.fori_loop(0, NCH, chunk, 0)
    plsc.subcore_barrier()

    pltpu.sync_copy(acc_sh.at[pl.ds(s * WPT, WPT)],
                    out.at[pl.ds(c * NPAD + s * WPT, WPT)])


@functools.cache
def _deg_call():
    return pl.kernel(
        _deg_body,
        out_type=jax.ShapeDtypeStruct((2 * NPAD,), jnp.float32),
        mesh=_mesh(),
        scratch_types=[
            pltpu.VMEM((NCH, K), jnp.int32),
            pltpu.VMEM((128,), jnp.float32),
            pltpu.VMEM((WPT,), jnp.float32),
            pltpu.VMEM_SHARED((NPAD,), jnp.float32),
        ],
    )


def _scatter_body(row3, col3, m, out, row_v, col_v, buf, acc_sh):
    c = lax.axis_index("c")
    s = lax.axis_index("s")
    wid = c * NS + s

    def zrow(i, carry):
        for j in range(D // 16):
            buf[i, pl.ds(16 * j, 16)] = jnp.zeros((16,), jnp.float32)
        return carry

    lax.fori_loop(0, ZR, zrow, 0)
    for i in range(RPT // ZR):
        pltpu.sync_copy(buf, acc_sh.at[pl.ds(s * RPT + i * ZR, ZR)])
    plsc.subcore_barrier()

    pltpu.sync_copy(row3.at[wid], row_v)
    pltpu.sync_copy(col3.at[wid], col_v)

    gbuf = buf.at[pl.ds(0, K)]

    def chunk(j, carry):
        pltpu.sync_copy(m.at[row_v.at[j]], gbuf)
        pltpu.sync_copy(gbuf, acc_sh.at[col_v.at[j]], add=True)
        return carry

    lax.fori_loop(0, NCH, chunk, 0)
    plsc.subcore_barrier()

    pltpu.sync_copy(acc_sh.at[pl.ds(s * RPT, RPT)],
                    out.at[pl.ds(c * NPAD + s * RPT, RPT)])


@functools.cache
def _scatter_call():
    return pl.kernel(
        _scatter_body,
        out_type=jax.ShapeDtypeStruct((2 * NPAD, D), jnp.float32),
        mesh=_mesh(),
        scratch_types=[
            pltpu.VMEM((NCH, K), jnp.int32),
            pltpu.VMEM((NCH, K), jnp.int32),
            pltpu.VMEM((ZR, D), jnp.float32),
            pltpu.VMEM_SHARED((NPAD, D), jnp.float32),
        ],
    )


def _dinv(deg_ref):
    deg = deg_ref[0] + deg_ref[1] + 1.0
    return lax.rsqrt(deg)


def _mm_scale_body(x_ref, w_ref, deg_ref, o_ref):
    h = jnp.dot(x_ref[...], w_ref[...], preferred_element_type=jnp.float32)
    o_ref[...] = _dinv(deg_ref) * h


def _layer_mid_body(s_ref, m_ref, deg_ref, b_ref, w_ref, o_ref):
    dinv = _dinv(deg_ref)
    h = dinv * (s_ref[0] + s_ref[1] + m_ref[...]) + b_ref[...]
    o_ref[...] = dinv * jnp.dot(h, w_ref[...],
                                preferred_element_type=jnp.float32)


def _pool_body(s_ref, m_ref, deg_ref, b_ref, batch_ref,
               o_ref, max_acc, sum_acc, cnt_acc):
    i = pl.program_id(0)

    @pl.when(i == 0)
    def _():
        max_acc[...] = jnp.full((G, D), -jnp.inf, jnp.float32)
        sum_acc[...] = jnp.zeros((G, D), jnp.float32)
        cnt_acc[...] = jnp.zeros((G, 1), jnp.float32)

    dinv = _dinv(deg_ref)
    h = dinv * (s_ref[0] + s_ref[1] + m_ref[...]) + b_ref[...]

    b = batch_ref[...]
    gids = lax.broadcasted_iota(jnp.int32, (1, G), 1)
    oh = (b == gids).astype(jnp.float32)
    sum_acc[...] += lax.dot_general(oh, h, (((0,), (0,)), ((), ())),
                                    preferred_element_type=jnp.float32)
    cnt_acc[...] += lax.dot_general(oh, jnp.ones((BLK, 1), jnp.float32),
                                    (((0,), (0,)), ((), ())),
                                    preferred_element_type=jnp.float32)

    b_lo = jnp.min(b)
    b_hi = jnp.max(b)
    for g in range(G):
        @pl.when((b_lo <= g) & (g <= b_hi))
        def _():
            masked = jnp.where(b == g, h, -jnp.inf)
            max_acc[g, :] = jnp.maximum(max_acc[g, :], jnp.max(masked, axis=0))

    @pl.when(i == GRID - 1)
    def _():
        cnt = cnt_acc[...]
        x_max = jnp.where(cnt > 0.0, max_acc[...], 0.0)
        x_mean = sum_acc[...] / jnp.maximum(cnt, 1.0)
        o_ref[:, pl.ds(0, D)] = x_max
        o_ref[:, pl.ds(D, D)] = x_mean


_row_spec = pl.BlockSpec((BLK, D), lambda i: (i, 0))
_col_spec = pl.BlockSpec((BLK, 1), lambda i: (i, 0))
_s_spec = pl.BlockSpec((2, BLK, D), lambda i: (0, i, 0))
_deg_spec = pl.BlockSpec((2, BLK, 1), lambda i: (0, i, 0))
_w_spec = pl.BlockSpec((D, D), lambda i: (0, 0))
_b_spec = pl.BlockSpec((1, D), lambda i: (0, 0))

_mm_scale_call = pl.pallas_call(
    _mm_scale_body,
    grid=(GRID,),
    in_specs=[_row_spec, _w_spec, _deg_spec],
    out_specs=_row_spec,
    out_shape=jax.ShapeDtypeStruct((N, D), jnp.float32),
)

_layer_mid_call = pl.pallas_call(
    _layer_mid_body,
    grid=(GRID,),
    in_specs=[_s_spec, _row_spec, _deg_spec, _b_spec, _w_spec],
    out_specs=_row_spec,
    out_shape=jax.ShapeDtypeStruct((N, D), jnp.float32),
)

_pool_call = pl.pallas_call(
    _pool_body,
    grid=(GRID,),
    in_specs=[_s_spec, _row_spec, _deg_spec, _b_spec, _col_spec],
    out_specs=pl.BlockSpec((G, 2 * D), lambda i: (0, 0)),
    out_shape=jax.ShapeDtypeStruct((G, 2 * D), jnp.float32),
    scratch_shapes=[pltpu.VMEM((G, D), jnp.float32),
                    pltpu.VMEM((G, D), jnp.float32),
                    pltpu.VMEM((G, 1), jnp.float32)],
)


def kernel(x, edge_index, batch, W1, b1, W2, b2):
    row3 = edge_index[0].reshape(NW, NCH, K)
    col3 = edge_index[1].reshape(NW, NCH, K)
    batch_c = batch.reshape(N, 1)
    b1r = b1.reshape(1, D)
    b2r = b2.reshape(1, D)

    deg = _deg_call()(col3).reshape(2, NPAD, 1)
    m1 = _mm_scale_call(x, W1, deg)
    s1 = _scatter_call()(row3, col3, m1).reshape(2, NPAD, D)
    m2 = _layer_mid_call(s1, m1, deg, b1r, W2)
    s2 = _scatter_call()(row3, col3, m2).reshape(2, NPAD, D)
    return _pool_call(s2, m2, deg, b2r, batch_c)


# speedup vs baseline: 29.1664x; 1.3053x over previous
import functools

import jax
import jax.numpy as jnp
from jax import lax
from jax.experimental import pallas as pl
from jax.experimental.pallas import tpu as pltpu
from jax.experimental.pallas import tpu_sc as plsc

N = 10000
NPAD = 10112
DPAD = 10240
D = 128
E = 320000
G = 64
NC, NS = 2, 16
NW = NC * NS
EPW = E // NW
K = 80
NCH = EPW // K
RPT = NPAD // NS
WPT = DPAD // NS
BLK = 1000
GRID = N // BLK

@functools.cache
def _mesh():
    return plsc.VectorSubcoreMesh(core_axis_name="c", subcore_axis_name="s",
                                  num_cores=NC, num_subcores=NS)


def _deg_body(col3, out, col_v, ones_v, zrow_v, acc_sh):
    c = lax.axis_index("c")
    s = lax.axis_index("s")
    wid = c * NS + s

    for i in range(8):
        ones_v[pl.ds(16 * i, 16)] = jnp.ones((16,), jnp.float32)
    for i in range(640 // 16):
        zrow_v[pl.ds(16 * i, 16)] = jnp.zeros((16,), jnp.float32)
    pltpu.sync_copy(zrow_v.at[pl.ds(0, WPT)], acc_sh.at[pl.ds(s * WPT, WPT)])
    plsc.subcore_barrier()

    pltpu.sync_copy(col3.at[wid], col_v)

    def chunk(j, carry):
        pltpu.sync_copy(ones_v.at[pl.ds(0, K)], acc_sh.at[col_v.at[j]],
                        add=True)
        return carry

    lax.fori_loop(0, NCH, chunk, 0)
    plsc.subcore_barrier()

    pltpu.sync_copy(acc_sh.at[pl.ds(s * WPT, WPT)],
                    out.at[pl.ds(c * DPAD + s * WPT, WPT)])


@functools.cache
def _deg_call():
    return pl.kernel(
        _deg_body,
        out_type=jax.ShapeDtypeStruct((2 * DPAD,), jnp.float32),
        mesh=_mesh(),
        scratch_types=[
            pltpu.VMEM((NCH, K), jnp.int32),
            pltpu.VMEM((128,), jnp.float32),
            pltpu.VMEM((640,), jnp.float32),
            pltpu.VMEM_SHARED((DPAD,), jnp.float32),
        ],
    )


def _scatter_body(row2, col3, m, zeros_h, out, row_v, col_v, buf0, buf1,
                  acc_sh, sem0, sem1):
    c = lax.axis_index("c")
    s = lax.axis_index("s")
    wid = c * NS + s

    pltpu.sync_copy(row2.at[wid], row_v)
    pltpu.sync_copy(col3.at[wid], col_v)

    def fire(j, buf, sem):
        pltpu.async_copy(m.at[row_v.at[pl.ds(j * K, K)]], buf, sem)

    def drain(buf, sem):
        pltpu.make_async_copy(zeros_h.at[pl.ds(0, K)], buf, sem).wait()

    fire(0, buf0, sem0)
    fire(1, buf1, sem1)
    pltpu.sync_copy(zeros_h.at[pl.ds(s * RPT, RPT)],
                    acc_sh.at[pl.ds(s * RPT, RPT)])
    plsc.subcore_barrier()

    def it(t, carry):
        j0 = 2 * t
        drain(buf0, sem0)
        pltpu.sync_copy(buf0, acc_sh.at[col_v.at[j0]], add=True)
        fire(j0 + 2, buf0, sem0)
        drain(buf1, sem1)
        pltpu.sync_copy(buf1, acc_sh.at[col_v.at[j0 + 1]], add=True)
        fire(j0 + 3, buf1, sem1)
        return carry

    lax.fori_loop(0, NCH // 2 - 1, it, 0)
    drain(buf0, sem0)
    pltpu.sync_copy(buf0, acc_sh.at[col_v.at[NCH - 3]], add=True)
    fire(NCH - 1, buf0, sem0)
    drain(buf1, sem1)
    pltpu.sync_copy(buf1, acc_sh.at[col_v.at[NCH - 2]], add=True)
    drain(buf0, sem0)
    pltpu.sync_copy(buf0, acc_sh.at[col_v.at[NCH - 1]], add=True)
    plsc.subcore_barrier()

    pltpu.sync_copy(acc_sh.at[pl.ds(s * RPT, RPT)],
                    out.at[pl.ds(c * NPAD + s * RPT, RPT)])


@functools.cache
def _scatter_call():
    return pl.kernel(
        _scatter_body,
        out_type=jax.ShapeDtypeStruct((2 * NPAD, D), jnp.float32),
        mesh=_mesh(),
        scratch_types=[
            pltpu.VMEM((EPW,), jnp.int32),
            pltpu.VMEM((NCH, K), jnp.int32),
            pltpu.VMEM((K, D), jnp.float32),
            pltpu.VMEM((K, D), jnp.float32),
            pltpu.VMEM_SHARED((NPAD, D), jnp.float32),
            pltpu.SemaphoreType.DMA,
            pltpu.SemaphoreType.DMA,
        ],
    )


def _dinv(deg_ref):
    deg = deg_ref[0] + deg_ref[1] + 1.0
    return lax.rsqrt(deg)


def _mm_scale_body(x_ref, w_ref, deg_ref, o_ref):
    h = jnp.dot(x_ref[...], w_ref[...], preferred_element_type=jnp.float32)
    o_ref[...] = _dinv(deg_ref) * h


def _layer_mid_body(s_ref, m_ref, deg_ref, b_ref, w_ref, o_ref):
    dinv = _dinv(deg_ref)
    h = dinv * (s_ref[0] + s_ref[1] + m_ref[...]) + b_ref[...]
    o_ref[...] = dinv * jnp.dot(h, w_ref[...],
                                preferred_element_type=jnp.float32)


def _pool_body(s_ref, m_ref, deg_ref, b_ref, batch_ref,
               o_ref, max_acc, sum_acc, cnt_acc):
    i = pl.program_id(0)

    @pl.when(i == 0)
    def _():
        max_acc[...] = jnp.full((G, D), -jnp.inf, jnp.float32)
        sum_acc[...] = jnp.zeros((G, D), jnp.float32)
        cnt_acc[...] = jnp.zeros((G, 1), jnp.float32)

    dinv = _dinv(deg_ref)
    h = dinv * (s_ref[0] + s_ref[1] + m_ref[...]) + b_ref[...]

    b = batch_ref[...]
    gids = lax.broadcasted_iota(jnp.int32, (1, G), 1)
    oh = (b == gids).astype(jnp.float32)
    sum_acc[...] += lax.dot_general(oh, h, (((0,), (0,)), ((), ())),
                                    preferred_element_type=jnp.float32)
    cnt_acc[...] += lax.dot_general(oh, jnp.ones((BLK, 1), jnp.float32),
                                    (((0,), (0,)), ((), ())),
                                    preferred_element_type=jnp.float32)

    b_lo = jnp.min(b)
    b_hi = jnp.max(b)
    for g in range(G):
        @pl.when((b_lo <= g) & (g <= b_hi))
        def _():
            masked = jnp.where(b == g, h, -jnp.inf)
            max_acc[g, :] = jnp.maximum(max_acc[g, :], jnp.max(masked, axis=0))

    @pl.when(i == GRID - 1)
    def _():
        cnt = cnt_acc[...]
        x_max = jnp.where(cnt > 0.0, max_acc[...], 0.0)
        x_mean = sum_acc[...] / jnp.maximum(cnt, 1.0)
        o_ref[:, pl.ds(0, D)] = x_max
        o_ref[:, pl.ds(D, D)] = x_mean


_row_spec = pl.BlockSpec((BLK, D), lambda i: (i, 0))
_col_spec = pl.BlockSpec((BLK, 1), lambda i: (i, 0))
_s_spec = pl.BlockSpec((2, BLK, D), lambda i: (0, i, 0))
_deg_spec = pl.BlockSpec((2, BLK, 1), lambda i: (0, i, 0))
_w_spec = pl.BlockSpec((D, D), lambda i: (0, 0))
_b_spec = pl.BlockSpec((1, D), lambda i: (0, 0))

_mm_scale_call = pl.pallas_call(
    _mm_scale_body,
    grid=(GRID,),
    in_specs=[_row_spec, _w_spec, _deg_spec],
    out_specs=_row_spec,
    out_shape=jax.ShapeDtypeStruct((N, D), jnp.float32),
)

_layer_mid_call = pl.pallas_call(
    _layer_mid_body,
    grid=(GRID,),
    in_specs=[_s_spec, _row_spec, _deg_spec, _b_spec, _w_spec],
    out_specs=_row_spec,
    out_shape=jax.ShapeDtypeStruct((N, D), jnp.float32),
)

_pool_call = pl.pallas_call(
    _pool_body,
    grid=(GRID,),
    in_specs=[_s_spec, _row_spec, _deg_spec, _b_spec, _col_spec],
    out_specs=pl.BlockSpec((G, 2 * D), lambda i: (0, 0)),
    out_shape=jax.ShapeDtypeStruct((G, 2 * D), jnp.float32),
    scratch_shapes=[pltpu.VMEM((G, D), jnp.float32),
                    pltpu.VMEM((G, D), jnp.float32),
                    pltpu.VMEM((G, 1), jnp.float32)],
)


def kernel(x, edge_index, batch, W1, b1, W2, b2):
    row2 = edge_index[0].reshape(NW, EPW)
    col3 = edge_index[1].reshape(NW, NCH, K)
    batch_c = batch.reshape(N, 1)
    b1r = b1.reshape(1, D)
    b2r = b2.reshape(1, D)

    zeros_h = jnp.zeros((NPAD, D), jnp.float32)
    deg = _deg_call()(col3).reshape(2, DPAD, 1)
    m1 = _mm_scale_call(x, W1, deg)
    s1 = _scatter_call()(row2, col3, m1, zeros_h).reshape(2, NPAD, D)
    m2 = _layer_mid_call(s1, m1, deg, b1r, W2)
    s2 = _scatter_call()(row2, col3, m2, zeros_h).reshape(2, NPAD, D)
    return _pool_call(s2, m2, deg, b2r, batch_c)
